# baseline (speedup 1.0000x reference)
import functools

import jax
import jax.numpy as jnp
import numpy as np
from jax import lax
from jax.experimental import pallas as pl
from jax.experimental.pallas import tpu as pltpu
from jax.experimental.pallas import tpu_sc as plsc

_B = 128
_V = 6890
_E = 20664
_NC = 2
_NS = 16
_NW = _NC * _NS
_C = 72
_K = 9
_WPW = _C * _K
_EPAD = _NW * _WPW
_NG = _B // 16
_MAGIC = 0x5F3759DF
_NEWTON = 1

_mesh = plsc.VectorSubcoreMesh(core_axis_name="c", subcore_axis_name="s")


@functools.partial(
    pl.kernel,
    mesh=_mesh,
    out_type=jax.ShapeDtypeStruct((_NW, _B), jnp.float32),
    scratch_types=[
        pltpu.VMEM((_WPW,), jnp.int32),
        pltpu.VMEM((_WPW,), jnp.int32),
        pltpu.VMEM((2, _C, _B), jnp.int32),
        pltpu.VMEM((2, _C, _B), jnp.float32),
        pltpu.VMEM((2, _C, _B), jnp.int32),
        pltpu.VMEM((2, _C, _B), jnp.float32),
        pltpu.VMEM((_B,), jnp.float32),
        pltpu.SemaphoreType.DMA,
        pltpu.SemaphoreType.DMA,
        pltpu.SemaphoreType.DMA,
        pltpu.SemaphoreType.DMA,
        pltpu.SemaphoreType.DMA,
        pltpu.SemaphoreType.DMA,
        pltpu.SemaphoreType.DMA,
        pltpu.SemaphoreType.DMA,
    ],
)
def _edge_len_partials(e0_hbm, e1_hbm, xy_hbm, z_hbm, out_hbm,
                       e0_v, e1_v, xy0, z0, xy1, z1, acc_v,
                       sa0, sa1, sb0, sb1, sc0, sc1, sd0, sd1):
    wid = lax.axis_index("s") * _NC + lax.axis_index("c")
    base = wid * _WPW
    pltpu.sync_copy(e0_hbm.at[pl.ds(base, _WPW)], e0_v)
    pltpu.sync_copy(e1_hbm.at[pl.ds(base, _WPW)], e1_v)

    sems = ((sa0, sa1), (sb0, sb1), (sc0, sc1), (sd0, sd1))

    def start(k):
        b = k % 2
        i0 = e0_v.at[pl.ds(k * _C, _C)]
        i1 = e1_v.at[pl.ds(k * _C, _C)]
        return (
            pltpu.async_copy(xy_hbm.at[i0], xy0.at[b], sems[0][b]),
            pltpu.async_copy(z_hbm.at[i0], z0.at[b], sems[1][b]),
            pltpu.async_copy(xy_hbm.at[i1], xy1.at[b], sems[2][b]),
            pltpu.async_copy(z_hbm.at[i1], z1.at[b], sems[3][b]),
        )

    pending = {0: start(0)}
    acc = tuple(jnp.zeros((16,), jnp.float32) for _ in range(_NG))

    half = jnp.full((16,), 0.5, jnp.float32)
    three_half = jnp.full((16,), 1.5, jnp.float32)
    magic = jnp.full((16,), _MAGIC, jnp.int32)
    sixteen = jnp.full((16,), 16, jnp.int32)
    himask = jnp.full((16,), -65536, jnp.int32)

    def sqrt_acc(a, ssq):
        yi = magic - lax.shift_right_logical(
            lax.bitcast_convert_type(ssq, jnp.int32), 1)
        y = lax.bitcast_convert_type(yi, jnp.float32)
        h = half * ssq
        for _ in range(_NEWTON):
            y = y * (three_half - h * y * y)
        return a + ssq * y

    for k in range(_K):
        if k + 1 < _K:
            pending[k + 1] = start(k + 1)
        for c in pending.pop(k):
            c.wait()
        b = k % 2
        rxy0, rz0, rxy1, rz1 = xy0.at[b], z0.at[b], xy1.at[b], z1.at[b]

        def edge(i, acc):
            out = []
            for g in range(_NG):
                o = g * 16
                w0 = rxy0[i, pl.ds(o, 16)]
                w1 = rxy1[i, pl.ds(o, 16)]
                x0 = lax.bitcast_convert_type(lax.shift_left(w0, sixteen),
                                              jnp.float32)
                y0 = lax.bitcast_convert_type(lax.bitwise_and(w0, himask),
                                              jnp.float32)
                x1 = lax.bitcast_convert_type(lax.shift_left(w1, sixteen),
                                              jnp.float32)
                y1 = lax.bitcast_convert_type(lax.bitwise_and(w1, himask),
                                              jnp.float32)
                dx = x0 - x1
                dy = y0 - y1
                dz = rz0[i, pl.ds(o, 16)] - rz1[i, pl.ds(o, 16)]
                ssq = dx * dx + dy * dy + dz * dz
                out.append(sqrt_acc(acc[g], ssq))
            return tuple(out)

        def body(p, acc):
            i = p * 2
            return edge(i + 1, edge(i, acc))

        acc = lax.fori_loop(0, _C // 2, body, acc)

    for g in range(_NG):
        acc_v[pl.ds(g * 16, 16)] = acc[g]
    pltpu.sync_copy(acc_v, out_hbm.at[wid])


def kernel(pred_vertices, has_smpl, edge):
    pv16 = pred_vertices.astype(jnp.bfloat16)
    xy = lax.bitcast_convert_type(pv16[:, :, :2], jnp.int32)
    xy_t = jnp.transpose(xy)
    z_t = jnp.transpose(pred_vertices[:, :, 2])

    pad = jnp.zeros((_EPAD - _E, 2), jnp.int32)
    ep = jnp.concatenate([edge, pad], axis=0)
    partials = _edge_len_partials(ep[:, 0], ep[:, 1], xy_t, z_t)

    mask = (has_smpl == 1).astype(jnp.float32)
    n_sel = jnp.sum(mask)
    per_b = jnp.sum(partials, axis=0)
    total = jnp.sum(per_b * mask)
    return total / (n_sel * _E)


# speedup vs baseline: 1.0667x; 1.0667x over previous
import functools

import jax
import jax.numpy as jnp
import numpy as np
from jax import lax
from jax.experimental import pallas as pl
from jax.experimental.pallas import tpu as pltpu
from jax.experimental.pallas import tpu_sc as plsc

_B = 128
_V = 6890
_E = 20664
_NC = 2
_NS = 16
_NW = _NC * _NS
_C = 72
_K = 9
_WPW = _C * _K
_EPAD = _NW * _WPW
_NG = _B // 16
_MAGIC = 0x5F3759DF
_NEWTON = 1

_mesh = plsc.VectorSubcoreMesh(core_axis_name="c", subcore_axis_name="s")


@functools.partial(
    pl.kernel,
    mesh=_mesh,
    out_type=jax.ShapeDtypeStruct((_NW, _B), jnp.float32),
    scratch_types=[
        pltpu.VMEM((_WPW,), jnp.int32),
        pltpu.VMEM((_WPW,), jnp.int32),
        pltpu.VMEM((2, _C, _B), jnp.int32),
        pltpu.VMEM((2, _C, _B), jnp.float32),
        pltpu.VMEM((2, _C, _B), jnp.int32),
        pltpu.VMEM((2, _C, _B), jnp.float32),
        pltpu.VMEM((_B,), jnp.float32),
        pltpu.SemaphoreType.DMA,
        pltpu.SemaphoreType.DMA,
        pltpu.SemaphoreType.DMA,
        pltpu.SemaphoreType.DMA,
        pltpu.SemaphoreType.DMA,
        pltpu.SemaphoreType.DMA,
        pltpu.SemaphoreType.DMA,
        pltpu.SemaphoreType.DMA,
    ],
)
def _edge_len_partials(e0_hbm, e1_hbm, xy_hbm, z_hbm, out_hbm,
                       e0_v, e1_v, xy0, z0, xy1, z1, acc_v,
                       sa0, sa1, sb0, sb1, sc0, sc1, sd0, sd1):
    wid = lax.axis_index("s") * _NC + lax.axis_index("c")
    base = wid * _WPW
    pltpu.sync_copy(e0_hbm.at[pl.ds(base, _WPW)], e0_v)
    pltpu.sync_copy(e1_hbm.at[pl.ds(base, _WPW)], e1_v)

    sems = ((sa0, sa1), (sb0, sb1), (sc0, sc1), (sd0, sd1))

    def start(k):
        b = k % 2
        i0 = e0_v.at[pl.ds(k * _C, _C)]
        i1 = e1_v.at[pl.ds(k * _C, _C)]
        return (
            pltpu.async_copy(xy_hbm.at[i0], xy0.at[b], sems[0][b]),
            pltpu.async_copy(z_hbm.at[i0], z0.at[b], sems[1][b]),
            pltpu.async_copy(xy_hbm.at[i1], xy1.at[b], sems[2][b]),
            pltpu.async_copy(z_hbm.at[i1], z1.at[b], sems[3][b]),
        )

    pending = {0: start(0)}
    acc = tuple(jnp.zeros((16,), jnp.float32) for _ in range(_NG))

    half = jnp.full((16,), 0.5, jnp.float32)
    three_half = jnp.full((16,), 1.5, jnp.float32)
    magic = jnp.full((16,), _MAGIC, jnp.int32)
    sixteen = jnp.full((16,), 16, jnp.int32)
    himask = jnp.full((16,), -65536, jnp.int32)

    def sqrt_acc(a, ssq):
        yi = magic - lax.shift_right_logical(
            lax.bitcast_convert_type(ssq, jnp.int32), 1)
        y = lax.bitcast_convert_type(yi, jnp.float32)
        h = half * ssq
        for _ in range(_NEWTON):
            y = y * (three_half - h * y * y)
        return a + ssq * y

    for k in range(_K):
        if k + 1 < _K:
            pending[k + 1] = start(k + 1)
        for c in pending.pop(k):
            c.wait()
        b = k % 2
        rxy0, rz0, rxy1, rz1 = xy0.at[b], z0.at[b], xy1.at[b], z1.at[b]

        def edge(i, acc):
            out = []
            for g in range(_NG):
                o = g * 16
                w0 = rxy0[i, pl.ds(o, 16)]
                w1 = rxy1[i, pl.ds(o, 16)]
                x0 = lax.bitcast_convert_type(lax.shift_left(w0, sixteen),
                                              jnp.float32)
                y0 = lax.bitcast_convert_type(lax.bitwise_and(w0, himask),
                                              jnp.float32)
                x1 = lax.bitcast_convert_type(lax.shift_left(w1, sixteen),
                                              jnp.float32)
                y1 = lax.bitcast_convert_type(lax.bitwise_and(w1, himask),
                                              jnp.float32)
                dx = x0 - x1
                dy = y0 - y1
                dz = rz0[i, pl.ds(o, 16)] - rz1[i, pl.ds(o, 16)]
                ssq = dx * dx + dy * dy + dz * dz
                out.append(sqrt_acc(acc[g], ssq))
            return tuple(out)

        acc = lax.fori_loop(0, _C, edge, acc)

    for g in range(_NG):
        acc_v[pl.ds(g * 16, 16)] = acc[g]
    pltpu.sync_copy(acc_v, out_hbm.at[wid])


def kernel(pred_vertices, has_smpl, edge):
    pv16 = pred_vertices.astype(jnp.bfloat16)
    xy = lax.bitcast_convert_type(pv16[:, :, :2], jnp.int32)
    xy_t = jnp.transpose(xy)
    z_t = jnp.transpose(pred_vertices[:, :, 2])

    pad = jnp.zeros((_EPAD - _E, 2), jnp.int32)
    ep = jnp.concatenate([edge, pad], axis=0)
    partials = _edge_len_partials(ep[:, 0], ep[:, 1], xy_t, z_t)

    mask = (has_smpl == 1).astype(jnp.float32)
    n_sel = jnp.sum(mask)
    per_b = jnp.sum(partials, axis=0)
    total = jnp.sum(per_b * mask)
    return total / (n_sel * _E)


# speedup vs baseline: 1.0672x; 1.0005x over previous
import functools

import jax
import jax.numpy as jnp
import numpy as np
from jax import lax
from jax.experimental import pallas as pl
from jax.experimental.pallas import tpu as pltpu
from jax.experimental.pallas import tpu_sc as plsc

_B = 128
_V = 6890
_E = 20664
_NC = 2
_NS = 16
_NW = _NC * _NS
_C = 72
_K = 9
_WPW = _C * _K
_EPAD = _NW * _WPW
_NG = _B // 16
_MAGIC = 0x5F3759DF
_NEWTON = 1

_mesh = plsc.VectorSubcoreMesh(core_axis_name="c", subcore_axis_name="s")


@functools.partial(
    pl.kernel,
    mesh=_mesh,
    out_type=jax.ShapeDtypeStruct((_NW, _B), jnp.float32),
    scratch_types=[
        pltpu.VMEM((_WPW,), jnp.int32),
        pltpu.VMEM((_WPW,), jnp.int32),
        pltpu.VMEM((2, _C, _B), jnp.int32),
        pltpu.VMEM((2, _C, _B), jnp.float32),
        pltpu.VMEM((2, _C, _B), jnp.int32),
        pltpu.VMEM((2, _C, _B), jnp.float32),
        pltpu.VMEM((_B,), jnp.float32),
        pltpu.SemaphoreType.DMA,
        pltpu.SemaphoreType.DMA,
        pltpu.SemaphoreType.DMA,
        pltpu.SemaphoreType.DMA,
        pltpu.SemaphoreType.DMA,
        pltpu.SemaphoreType.DMA,
        pltpu.SemaphoreType.DMA,
        pltpu.SemaphoreType.DMA,
    ],
)
def _edge_len_partials(e0_hbm, e1_hbm, xy_hbm, z_hbm, out_hbm,
                       e0_v, e1_v, xy0, z0, xy1, z1, acc_v,
                       sa0, sa1, sb0, sb1, sc0, sc1, sd0, sd1):
    wid = lax.axis_index("s") * _NC + lax.axis_index("c")
    base = wid * _WPW
    pltpu.sync_copy(e0_hbm.at[pl.ds(base, _WPW)], e0_v)
    pltpu.sync_copy(e1_hbm.at[pl.ds(base, _WPW)], e1_v)

    sems = ((sa0, sa1), (sb0, sb1), (sc0, sc1), (sd0, sd1))

    def start(k):
        b = k % 2
        i0 = e0_v.at[pl.ds(k * _C, _C)]
        i1 = e1_v.at[pl.ds(k * _C, _C)]
        return (
            pltpu.async_copy(xy_hbm.at[i0], xy0.at[b], sems[0][b]),
            pltpu.async_copy(z_hbm.at[i0], z0.at[b], sems[1][b]),
            pltpu.async_copy(xy_hbm.at[i1], xy1.at[b], sems[2][b]),
            pltpu.async_copy(z_hbm.at[i1], z1.at[b], sems[3][b]),
        )

    pending = {0: start(0)}
    acc = tuple(jnp.zeros((16,), jnp.float32) for _ in range(_NG))

    half = jnp.full((16,), 0.5, jnp.float32)
    three_half = jnp.full((16,), 1.5, jnp.float32)
    magic = jnp.full((16,), _MAGIC, jnp.int32)
    sixteen = jnp.full((16,), 16, jnp.int32)
    himask = jnp.full((16,), -65536, jnp.int32)

    def sqrt_acc(a, ssq):
        yi = magic - lax.shift_right_logical(
            lax.bitcast_convert_type(ssq, jnp.int32), 1)
        y = lax.bitcast_convert_type(yi, jnp.float32)
        h = half * ssq
        for _ in range(_NEWTON):
            y = y * (three_half - h * y * y)
        return a + ssq * y

    for k in range(_K):
        if k + 1 < _K:
            pending[k + 1] = start(k + 1)
        for c in pending.pop(k):
            c.wait()
        b = k % 2
        rxy0, rz0, rxy1, rz1 = xy0.at[b], z0.at[b], xy1.at[b], z1.at[b]

        def edge(i, acc):
            out = []
            for g in range(_NG):
                o = g * 16
                w0 = rxy0[i, pl.ds(o, 16)]
                w1 = rxy1[i, pl.ds(o, 16)]
                x0 = lax.bitcast_convert_type(lax.shift_left(w0, sixteen),
                                              jnp.float32)
                y0 = lax.bitcast_convert_type(lax.bitwise_and(w0, himask),
                                              jnp.float32)
                x1 = lax.bitcast_convert_type(lax.shift_left(w1, sixteen),
                                              jnp.float32)
                y1 = lax.bitcast_convert_type(lax.bitwise_and(w1, himask),
                                              jnp.float32)
                dx = x0 - x1
                dy = y0 - y1
                dz = rz0[i, pl.ds(o, 16)] - rz1[i, pl.ds(o, 16)]
                ssq = dx * dx + dy * dy + dz * dz
                out.append(sqrt_acc(acc[g], ssq))
            return tuple(out)

        acc = plsc.parallel_loop(0, _C, 1, unroll=2, carry=acc)(edge)

    for g in range(_NG):
        acc_v[pl.ds(g * 16, 16)] = acc[g]
    pltpu.sync_copy(acc_v, out_hbm.at[wid])


def kernel(pred_vertices, has_smpl, edge):
    pv16 = pred_vertices.astype(jnp.bfloat16)
    xy = lax.bitcast_convert_type(pv16[:, :, :2], jnp.int32)
    xy_t = jnp.transpose(xy)
    z_t = jnp.transpose(pred_vertices[:, :, 2])

    pad = jnp.zeros((_EPAD - _E, 2), jnp.int32)
    ep = jnp.concatenate([edge, pad], axis=0)
    partials = _edge_len_partials(ep[:, 0], ep[:, 1], xy_t, z_t)

    mask = (has_smpl == 1).astype(jnp.float32)
    n_sel = jnp.sum(mask)
    per_b = jnp.sum(partials, axis=0)
    total = jnp.sum(per_b * mask)
    return total / (n_sel * _E)


# speedup vs baseline: 1.1077x; 1.0380x over previous
import functools

import jax
import jax.numpy as jnp
import numpy as np
from jax import lax
from jax.experimental import pallas as pl
from jax.experimental.pallas import tpu as pltpu
from jax.experimental.pallas import tpu_sc as plsc

_B = 128
_V = 6890
_E = 20664
_NC = 2
_NS = 16
_NW = _NC * _NS
_C = 72
_K = 9
_WPW = _C * _K
_EPAD = _NW * _WPW
_NG = _B // 16
_MAGIC = 0x5F3759DF
_NEWTON = 1

_mesh = plsc.VectorSubcoreMesh(core_axis_name="c", subcore_axis_name="s")


@functools.partial(
    pl.kernel,
    mesh=_mesh,
    out_type=jax.ShapeDtypeStruct((_NW, _B), jnp.float32),
    scratch_types=[
        pltpu.VMEM((_WPW,), jnp.int32),
        pltpu.VMEM((_WPW,), jnp.int32),
        pltpu.VMEM((3, _C, _B), jnp.int32),
        pltpu.VMEM((3, _C, _B), jnp.float32),
        pltpu.VMEM((3, _C, _B), jnp.int32),
        pltpu.VMEM((3, _C, _B), jnp.float32),
        pltpu.VMEM((_B,), jnp.float32),
        pltpu.SemaphoreType.DMA,
        pltpu.SemaphoreType.DMA,
        pltpu.SemaphoreType.DMA,
        pltpu.SemaphoreType.DMA,
        pltpu.SemaphoreType.DMA,
        pltpu.SemaphoreType.DMA,
        pltpu.SemaphoreType.DMA,
        pltpu.SemaphoreType.DMA,
        pltpu.SemaphoreType.DMA,
        pltpu.SemaphoreType.DMA,
        pltpu.SemaphoreType.DMA,
        pltpu.SemaphoreType.DMA,
    ],
)
def _edge_len_partials(e0_hbm, e1_hbm, xy_hbm, z_hbm, out_hbm,
                       e0_v, e1_v, xy0, z0, xy1, z1, acc_v,
                       sa0, sa1, sa2, sb0, sb1, sb2,
                       sc0, sc1, sc2, sd0, sd1, sd2):
    wid = lax.axis_index("s") * _NC + lax.axis_index("c")
    base = wid * _WPW
    pltpu.sync_copy(e0_hbm.at[pl.ds(base, _WPW)], e0_v)
    pltpu.sync_copy(e1_hbm.at[pl.ds(base, _WPW)], e1_v)

    sems = ((sa0, sa1, sa2), (sb0, sb1, sb2), (sc0, sc1, sc2), (sd0, sd1, sd2))

    def start(k):
        b = k % 3
        i0 = e0_v.at[pl.ds(k * _C, _C)]
        i1 = e1_v.at[pl.ds(k * _C, _C)]
        return (
            pltpu.async_copy(xy_hbm.at[i0], xy0.at[b], sems[0][b]),
            pltpu.async_copy(z_hbm.at[i0], z0.at[b], sems[1][b]),
            pltpu.async_copy(xy_hbm.at[i1], xy1.at[b], sems[2][b]),
            pltpu.async_copy(z_hbm.at[i1], z1.at[b], sems[3][b]),
        )

    pending = {0: start(0), 1: start(1)}
    acc = tuple(jnp.zeros((16,), jnp.float32) for _ in range(_NG))

    half = jnp.full((16,), 0.5, jnp.float32)
    three_half = jnp.full((16,), 1.5, jnp.float32)
    magic = jnp.full((16,), _MAGIC, jnp.int32)
    sixteen = jnp.full((16,), 16, jnp.int32)
    himask = jnp.full((16,), -65536, jnp.int32)

    def sqrt_acc(a, ssq):
        yi = magic - lax.shift_right_logical(
            lax.bitcast_convert_type(ssq, jnp.int32), 1)
        y = lax.bitcast_convert_type(yi, jnp.float32)
        h = half * ssq
        for _ in range(_NEWTON):
            y = y * (three_half - h * y * y)
        return a + ssq * y

    for k in range(_K):
        if k + 2 < _K:
            pending[k + 2] = start(k + 2)
        for c in pending.pop(k):
            c.wait()
        b = k % 3
        rxy0, rz0, rxy1, rz1 = xy0.at[b], z0.at[b], xy1.at[b], z1.at[b]

        def edge(i, acc):
            out = []
            for g in range(_NG):
                o = g * 16
                w0 = rxy0[i, pl.ds(o, 16)]
                w1 = rxy1[i, pl.ds(o, 16)]
                x0 = lax.bitcast_convert_type(lax.shift_left(w0, sixteen),
                                              jnp.float32)
                y0 = lax.bitcast_convert_type(lax.bitwise_and(w0, himask),
                                              jnp.float32)
                x1 = lax.bitcast_convert_type(lax.shift_left(w1, sixteen),
                                              jnp.float32)
                y1 = lax.bitcast_convert_type(lax.bitwise_and(w1, himask),
                                              jnp.float32)
                dx = x0 - x1
                dy = y0 - y1
                dz = rz0[i, pl.ds(o, 16)] - rz1[i, pl.ds(o, 16)]
                ssq = dx * dx + dy * dy + dz * dz
                out.append(sqrt_acc(acc[g], ssq))
            return tuple(out)

        acc = plsc.parallel_loop(0, _C, 1, unroll=2, carry=acc)(edge)

    for g in range(_NG):
        acc_v[pl.ds(g * 16, 16)] = acc[g]
    pltpu.sync_copy(acc_v, out_hbm.at[wid])


def kernel(pred_vertices, has_smpl, edge):
    pv16 = pred_vertices.astype(jnp.bfloat16)
    xy = lax.bitcast_convert_type(pv16[:, :, :2], jnp.int32)
    xy_t = jnp.transpose(xy)
    z_t = jnp.transpose(pred_vertices[:, :, 2])

    pad = jnp.zeros((_EPAD - _E, 2), jnp.int32)
    ep = jnp.concatenate([edge, pad], axis=0)
    partials = _edge_len_partials(ep[:, 0], ep[:, 1], xy_t, z_t)

    mask = (has_smpl == 1).astype(jnp.float32)
    n_sel = jnp.sum(mask)
    per_b = jnp.sum(partials, axis=0)
    total = jnp.sum(per_b * mask)
    return total / (n_sel * _E)


# speedup vs baseline: 1.1085x; 1.0007x over previous
import functools

import jax
import jax.numpy as jnp
import numpy as np
from jax import lax
from jax.experimental import pallas as pl
from jax.experimental.pallas import tpu as pltpu
from jax.experimental.pallas import tpu_sc as plsc

_B = 128
_V = 6890
_E = 20664
_NC = 2
_NS = 16
_NW = _NC * _NS
_C = 72
_K = 9
_WPW = _C * _K
_EPAD = _NW * _WPW
_NG = _B // 16
_MAGIC = 0x5F3759DF
_NEWTON = 1

_mesh = plsc.VectorSubcoreMesh(core_axis_name="c", subcore_axis_name="s")


@functools.partial(
    pl.kernel,
    mesh=_mesh,
    out_type=jax.ShapeDtypeStruct((_NW, _B), jnp.float32),
    scratch_types=[
        pltpu.VMEM((_WPW,), jnp.int32),
        pltpu.VMEM((_WPW,), jnp.int32),
        pltpu.VMEM((3, _C, _B), jnp.int32),
        pltpu.VMEM((3, _C, _B), jnp.float32),
        pltpu.VMEM((3, _C, _B), jnp.int32),
        pltpu.VMEM((3, _C, _B), jnp.float32),
        pltpu.VMEM((_B,), jnp.float32),
        pltpu.SemaphoreType.DMA,
        pltpu.SemaphoreType.DMA,
        pltpu.SemaphoreType.DMA,
        pltpu.SemaphoreType.DMA,
        pltpu.SemaphoreType.DMA,
        pltpu.SemaphoreType.DMA,
        pltpu.SemaphoreType.DMA,
        pltpu.SemaphoreType.DMA,
        pltpu.SemaphoreType.DMA,
        pltpu.SemaphoreType.DMA,
        pltpu.SemaphoreType.DMA,
        pltpu.SemaphoreType.DMA,
    ],
)
def _edge_len_partials(e0_hbm, e1_hbm, xy_hbm, z_hbm, out_hbm,
                       e0_v, e1_v, xy0, z0, xy1, z1, acc_v,
                       sa0, sa1, sa2, sb0, sb1, sb2,
                       sc0, sc1, sc2, sd0, sd1, sd2):
    wid = lax.axis_index("s") * _NC + lax.axis_index("c")
    base = wid * _WPW
    pltpu.sync_copy(e0_hbm.at[pl.ds(base, _WPW)], e0_v)
    pltpu.sync_copy(e1_hbm.at[pl.ds(base, _WPW)], e1_v)

    sems = ((sa0, sa1, sa2), (sb0, sb1, sb2), (sc0, sc1, sc2), (sd0, sd1, sd2))

    def start(k):
        b = k % 3
        i0 = e0_v.at[pl.ds(k * _C, _C)]
        i1 = e1_v.at[pl.ds(k * _C, _C)]
        return (
            pltpu.async_copy(xy_hbm.at[i0], xy0.at[b], sems[0][b]),
            pltpu.async_copy(z_hbm.at[i0], z0.at[b], sems[1][b]),
            pltpu.async_copy(xy_hbm.at[i1], xy1.at[b], sems[2][b]),
            pltpu.async_copy(z_hbm.at[i1], z1.at[b], sems[3][b]),
        )

    pending = {0: start(0), 1: start(1)}
    acc = tuple(jnp.zeros((16,), jnp.float32) for _ in range(_NG))

    half = jnp.full((16,), 0.5, jnp.float32)
    three_half = jnp.full((16,), 1.5, jnp.float32)
    magic = jnp.full((16,), _MAGIC, jnp.int32)
    sixteen = jnp.full((16,), 16, jnp.int32)
    himask = jnp.full((16,), -65536, jnp.int32)

    def sqrt_acc(a, ssq):
        yi = magic - lax.shift_right_logical(
            lax.bitcast_convert_type(ssq, jnp.int32), 1)
        y = lax.bitcast_convert_type(yi, jnp.float32)
        h = half * ssq
        for _ in range(_NEWTON):
            y = y * (three_half - h * y * y)
        return a + ssq * y

    for k in range(_K):
        if k + 2 < _K:
            pending[k + 2] = start(k + 2)
        for c in pending.pop(k):
            c.wait()
        b = k % 3
        rxy0, rz0, rxy1, rz1 = xy0.at[b], z0.at[b], xy1.at[b], z1.at[b]

        def edge(i, acc):
            out = []
            for g in range(_NG):
                o = g * 16
                w0 = rxy0[i, pl.ds(o, 16)]
                w1 = rxy1[i, pl.ds(o, 16)]
                x0 = lax.bitcast_convert_type(lax.shift_left(w0, sixteen),
                                              jnp.float32)
                y0 = lax.bitcast_convert_type(lax.bitwise_and(w0, himask),
                                              jnp.float32)
                x1 = lax.bitcast_convert_type(lax.shift_left(w1, sixteen),
                                              jnp.float32)
                y1 = lax.bitcast_convert_type(lax.bitwise_and(w1, himask),
                                              jnp.float32)
                dx = x0 - x1
                dy = y0 - y1
                dz = rz0[i, pl.ds(o, 16)] - rz1[i, pl.ds(o, 16)]
                ssq = dx * dx + dy * dy + dz * dz
                out.append(sqrt_acc(acc[g], ssq))
            return tuple(out)

        acc = plsc.parallel_loop(0, _C, 1, unroll=4, carry=acc)(edge)

    for g in range(_NG):
        acc_v[pl.ds(g * 16, 16)] = acc[g]
    pltpu.sync_copy(acc_v, out_hbm.at[wid])


def kernel(pred_vertices, has_smpl, edge):
    pv16 = pred_vertices.astype(jnp.bfloat16)
    xy = lax.bitcast_convert_type(pv16[:, :, :2], jnp.int32)
    xy_t = jnp.transpose(xy)
    z_t = jnp.transpose(pred_vertices[:, :, 2])

    pad = jnp.zeros((_EPAD - _E, 2), jnp.int32)
    ep = jnp.concatenate([edge, pad], axis=0)
    partials = _edge_len_partials(ep[:, 0], ep[:, 1], xy_t, z_t)

    mask = (has_smpl == 1).astype(jnp.float32)
    n_sel = jnp.sum(mask)
    per_b = jnp.sum(partials, axis=0)
    total = jnp.sum(per_b * mask)
    return total / (n_sel * _E)
